# baseline (speedup 1.0000x reference)
import functools

import jax
import jax.numpy as jnp
from jax import lax
from jax.experimental import pallas as pl
from jax.experimental.pallas import tpu as pltpu
from jax.experimental.pallas import tpu_sc as plsc

N = 10000
E = 320000
D = 128
DE = 16
H = 128
G = 64

_info = plsc.get_sparse_core_info()
NC = _info.num_cores
NS = _info.num_subcores
NW = NC * NS
EPW = E // NW
CH = 80
NCHUNK = EPW // CH
ZR = 40
WTILES = 10
RPS = N // WTILES

_mesh = plsc.VectorSubcoreMesh(core_axis_name="c", subcore_axis_name="s")


def _tables_body(x_ref, wd_ref, ws_ref, b_ref, td_ref, ts_ref):
    xb = x_ref[...]
    td_ref[...] = (
        jnp.dot(xb, wd_ref[...], preferred_element_type=jnp.float32) + b_ref[...]
    ).astype(jnp.bfloat16)
    ts_ref[...] = jnp.dot(
        xb, ws_ref[...], preferred_element_type=jnp.float32
    ).astype(jnp.bfloat16)


def _tables(x, wd, wsr, bcat):
    bm = 1000
    return pl.pallas_call(
        _tables_body,
        grid=(N // bm,),
        in_specs=[
            pl.BlockSpec((bm, D), lambda i: (i, 0)),
            pl.BlockSpec((D, 2 * D), lambda i: (0, 0)),
            pl.BlockSpec((D, 2 * D), lambda i: (0, 0)),
            pl.BlockSpec((1, 2 * D), lambda i: (0, 0)),
        ],
        out_specs=[
            pl.BlockSpec((bm, 2 * D), lambda i: (i, 0)),
            pl.BlockSpec((bm, 2 * D), lambda i: (i, 0)),
        ],
        out_shape=[
            jax.ShapeDtypeStruct((N, 2 * D), jnp.bfloat16),
            jax.ShapeDtypeStruct((N, 2 * D), jnp.bfloat16),
        ],
    )(x, wd, wsr, bcat)


@functools.partial(
    pl.kernel,
    out_type=jax.ShapeDtypeStruct((E, D), jnp.int32),
    mesh=_mesh,
    scratch_types=[
        pltpu.VMEM((NCHUNK, CH), jnp.int32),
        pltpu.VMEM((NCHUNK, CH), jnp.int32),
        pltpu.VMEM((CH, D), jnp.int32),
        pltpu.VMEM((CH, D), jnp.int32),
        pltpu.SemaphoreType.DMA,
        pltpu.SemaphoreType.DMA,
    ],
)
def _gather_k(td_hbm, ts_hbm, dst_hbm, src_hbm, g_hbm, idxd, idxs, bufd, bufs,
              semd, sems):
    wid = lax.axis_index("s") * NC + lax.axis_index("c")
    pltpu.sync_copy(dst_hbm.at[wid], idxd)
    pltpu.sync_copy(src_hbm.at[wid], idxs)

    def chunk(c, _):
        cpd = pltpu.make_async_copy(td_hbm.at[idxd.at[c]], bufd, semd)
        cps = pltpu.make_async_copy(ts_hbm.at[idxs.at[c]], bufs, sems)
        cpd.start()
        cps.start()
        cpd.wait()
        cps.wait()

        def add_row(r, _):
            msk = jnp.int32(-65536)
            for j in range(D // 16):
                s = pl.ds(j * 16, 16)
                wa = bufd[r, s]
                wb = bufs[r, s]
                flo = lax.bitcast_convert_type(
                    wa << 16, jnp.float32
                ) + lax.bitcast_convert_type(wb << 16, jnp.float32)
                fhi = lax.bitcast_convert_type(
                    wa & msk, jnp.float32
                ) + lax.bitcast_convert_type(wb & msk, jnp.float32)
                lo = lax.shift_right_logical(
                    lax.bitcast_convert_type(flo, jnp.int32), 16
                )
                hi = lax.bitcast_convert_type(fhi, jnp.int32) & msk
                bufd[r, s] = lo | hi
            return 0

        lax.fori_loop(0, CH, add_row, 0, unroll=2)
        pltpu.sync_copy(bufd, g_hbm.at[pl.ds(wid * EPW + c * CH, CH)])
        return 0

    lax.fori_loop(0, NCHUNK, chunk, 0)


def _msg_body(g_ref, ea_ref, we_ref, msg_ref):
    w = g_ref[...]
    gf = lax.bitcast_convert_type(w << 16, jnp.float32)
    gs = lax.bitcast_convert_type(w & jnp.int32(-65536), jnp.float32)
    ew = jnp.dot(ea_ref[...], we_ref[...], preferred_element_type=jnp.float32)
    msg_ref[...] = jax.nn.sigmoid(gf + ew[:, :D]) * jax.nn.softplus(
        gs + ew[:, D:]
    )


def _edge_msg(g, ea, we):
    bm = 2000
    return pl.pallas_call(
        _msg_body,
        grid=(E // bm,),
        in_specs=[
            pl.BlockSpec((bm, D), lambda i: (i, 0)),
            pl.BlockSpec((bm, DE), lambda i: (i, 0)),
            pl.BlockSpec((DE, 2 * D), lambda i: (0, 0)),
        ],
        out_specs=pl.BlockSpec((bm, D), lambda i: (i, 0)),
        out_shape=jax.ShapeDtypeStruct((E, D), jnp.float32),
    )(g, ea, we)


@functools.partial(
    pl.kernel,
    out_type=jax.ShapeDtypeStruct((NC, N, D), jnp.float32),
    mesh=_mesh,
    scratch_types=[
        pltpu.VMEM((NCHUNK, CH), jnp.int32),
        pltpu.VMEM((CH, D), jnp.float32),
        pltpu.VMEM((ZR, D), jnp.float32),
        pltpu.VMEM_SHARED((N, D), jnp.float32),
        pltpu.SemaphoreType.DMA,
    ],
)
def _scatter_k(msg_hbm, dst_hbm, aggp_hbm, idxd, mbuf, zbuf, aggsh, sem):
    cid = lax.axis_index("c")
    sid = lax.axis_index("s")
    wid = sid * NC + cid
    pltpu.sync_copy(dst_hbm.at[wid], idxd)

    def zrow(i, _):
        for j in range(D // 16):
            zbuf[i, pl.ds(j * 16, 16)] = jnp.zeros((16,), jnp.float32)
        return 0

    lax.fori_loop(0, ZR, zrow, 0)

    @pl.when(sid < WTILES)
    def _init():
        for t in range(RPS // ZR):
            pltpu.sync_copy(zbuf, aggsh.at[pl.ds(sid * RPS + t * ZR, ZR)])

    plsc.subcore_barrier()

    def chunk(c, _):
        pltpu.sync_copy(msg_hbm.at[pl.ds(wid * EPW + c * CH, CH)], mbuf)
        pltpu.sync_copy(mbuf, aggsh.at[idxd.at[c]], add=True)
        return 0

    lax.fori_loop(0, NCHUNK, chunk, 0)
    plsc.subcore_barrier()

    @pl.when(sid < WTILES)
    def _writeout():
        pltpu.sync_copy(
            aggsh.at[pl.ds(sid * RPS, RPS)],
            aggp_hbm.at[cid, pl.ds(sid * RPS, RPS)],
        )


def _final_body(x_ref, ap_ref, b_ref, w1_ref, b1_ref, w2_ref, b2_ref, out_ref):
    h = jax.nn.sigmoid(x_ref[...] + ap_ref[0] + ap_ref[1])
    h = jax.nn.sigmoid(
        jnp.dot(h, w1_ref[...], preferred_element_type=jnp.float32) + b1_ref[...]
    )
    oh = (
        b_ref[...] == lax.broadcasted_iota(jnp.int32, (N, G), 1)
    ).astype(jnp.float32)
    pooled = lax.dot_general(
        oh, h, (((0,), (0,)), ((), ())), preferred_element_type=jnp.float32
    )
    out = jnp.dot(pooled, w2_ref[...], preferred_element_type=jnp.float32)
    out_ref[...] = jnp.maximum(out + b2_ref[...], 0.0)


def _final(x, aggp, batch2d, w1, b1, w2, b2):
    return pl.pallas_call(
        _final_body,
        out_shape=jax.ShapeDtypeStruct((G, 1), jnp.float32),
    )(x, aggp, batch2d, w1, b1, w2, b2)


def kernel(x, edge_index, edge_attr, batch, Wf, bf, Ws, bs, W1, b1, W2, b2):
    src = edge_index[0]
    dst = edge_index[1]
    wd = jnp.concatenate([Wf[:D], Ws[:D]], axis=1)
    wsr = jnp.concatenate([Wf[D : 2 * D], Ws[D : 2 * D]], axis=1)
    we = jnp.concatenate([Wf[2 * D :], Ws[2 * D :]], axis=1)
    bcat = jnp.concatenate([bf, bs]).reshape(1, 2 * D)
    td, ts = _tables(x, wd, wsr, bcat)
    tdp = lax.bitcast_convert_type(
        jnp.stack([td[:, :D], td[:, D:]], axis=-1), jnp.int32
    )
    tsp = lax.bitcast_convert_type(
        jnp.stack([ts[:, :D], ts[:, D:]], axis=-1), jnp.int32
    )
    dst3 = dst.reshape(NW, NCHUNK, CH)
    src3 = src.reshape(NW, NCHUNK, CH)
    g = _gather_k(tdp, tsp, dst3, src3)
    msg = _edge_msg(g, edge_attr, we)
    aggp = _scatter_k(msg, dst3)
    return _final(
        x,
        aggp,
        batch.reshape(N, 1),
        W1,
        b1.reshape(1, H),
        W2,
        b2.reshape(1, 1),
    )


# speedup vs baseline: 1.4616x; 1.4616x over previous
import functools

import jax
import jax.numpy as jnp
from jax import lax
from jax.experimental import pallas as pl
from jax.experimental.pallas import tpu as pltpu
from jax.experimental.pallas import tpu_sc as plsc

N = 10000
E = 320000
D = 128
DE = 16
H = 128
G = 64

_info = plsc.get_sparse_core_info()
NC = _info.num_cores
NS = _info.num_subcores
NW = NC * NS
EPW = E // NW
CH = 40
NCHUNK = EPW // CH
CHS = 80
NCHS = EPW // CHS
ZR = 40
WTILES = 10
RPS = N // WTILES

_mesh = plsc.VectorSubcoreMesh(core_axis_name="c", subcore_axis_name="s")


def _tables_body(x_ref, wd_ref, ws_ref, b_ref, td_ref, ts_ref):
    xb = x_ref[...]
    td_ref[...] = (
        jnp.dot(xb, wd_ref[...], preferred_element_type=jnp.float32) + b_ref[...]
    ).astype(jnp.bfloat16)
    ts_ref[...] = jnp.dot(
        xb, ws_ref[...], preferred_element_type=jnp.float32
    ).astype(jnp.bfloat16)


def _tables(x, wd, wsr, bcat):
    bm = 1000
    return pl.pallas_call(
        _tables_body,
        grid=(N // bm,),
        in_specs=[
            pl.BlockSpec((bm, D), lambda i: (i, 0)),
            pl.BlockSpec((D, 2 * D), lambda i: (0, 0)),
            pl.BlockSpec((D, 2 * D), lambda i: (0, 0)),
            pl.BlockSpec((1, 2 * D), lambda i: (0, 0)),
        ],
        out_specs=[
            pl.BlockSpec((bm, 2 * D), lambda i: (i, 0)),
            pl.BlockSpec((bm, 2 * D), lambda i: (i, 0)),
        ],
        out_shape=[
            jax.ShapeDtypeStruct((N, 2 * D), jnp.bfloat16),
            jax.ShapeDtypeStruct((N, 2 * D), jnp.bfloat16),
        ],
    )(x, wd, wsr, bcat)


NBUF = 5
AHEAD = 2


@functools.partial(
    pl.kernel,
    out_type=[
        jax.ShapeDtypeStruct((E, D), jnp.int32),
        jax.ShapeDtypeStruct((E, D), jnp.int32),
    ],
    mesh=_mesh,
    scratch_types=[
        pltpu.VMEM((NCHUNK, CH), jnp.int32),
        pltpu.VMEM((NCHUNK, CH), jnp.int32),
        pltpu.VMEM((NBUF, CH, D), jnp.int32),
        pltpu.VMEM((NBUF, CH, D), jnp.int32),
        pltpu.SemaphoreType.DMA((NBUF,)),
        pltpu.SemaphoreType.DMA((NBUF,)),
    ],
)
def _gather_k(td_hbm, ts_hbm, dst_hbm, src_hbm, gd_hbm, gs_hbm, idxd, idxs,
              bufd, bufs, sg, sw):
    wid = lax.axis_index("s") * NC + lax.axis_index("c")
    pltpu.sync_copy(dst_hbm.at[wid], idxd)
    pltpu.sync_copy(src_hbm.at[wid], idxs)
    base = wid * EPW

    def gathers(c, slot):
        return (
            pltpu.make_async_copy(td_hbm.at[idxd.at[c]], bufd.at[slot],
                                  sg.at[slot]),
            pltpu.make_async_copy(ts_hbm.at[idxs.at[c]], bufs.at[slot],
                                  sg.at[slot]),
        )

    def writes(c, slot):
        rows = pl.ds(base + c * CH, CH)
        return (
            pltpu.make_async_copy(bufd.at[slot], gd_hbm.at[rows], sw.at[slot]),
            pltpu.make_async_copy(bufs.at[slot], gs_hbm.at[rows], sw.at[slot]),
        )

    for i in range(AHEAD):
        ga, gb = gathers(i, i)
        ga.start()
        gb.start()

    def outer(cc, _):
        for i in range(NBUF):
            c = cc * NBUF + i
            ga, gb = gathers(c, i)
            ga.wait()
            gb.wait()
            wa, wb = writes(c, i)
            wa.start()
            wb.start()
            j = (i + AHEAD) % NBUF
            cn = c + AHEAD

            @pl.when(cn >= NBUF)
            def _drain():
                pa, pb = writes(cn - NBUF, j)
                pa.wait()
                pb.wait()

            @pl.when(cn < NCHUNK)
            def _prefetch():
                na, nb = gathers(cn, j)
                na.start()
                nb.start()
        return 0

    lax.fori_loop(0, NCHUNK // NBUF, outer, 0)
    for k in range(NBUF - AHEAD):
        c = NCHUNK - (NBUF - AHEAD) + k
        wa, wb = writes(c, c % NBUF)
        wa.wait()
        wb.wait()


def _msg_body(gd_ref, gs_ref, ea_ref, we_ref, msg_ref):
    msk = jnp.int32(-65536)
    wd_ = gd_ref[...]
    ws_ = gs_ref[...]
    f = lax.bitcast_convert_type(wd_ << 16, jnp.float32) + lax.bitcast_convert_type(
        ws_ << 16, jnp.float32
    )
    s = lax.bitcast_convert_type(wd_ & msk, jnp.float32) + lax.bitcast_convert_type(
        ws_ & msk, jnp.float32
    )
    ew = jnp.dot(ea_ref[...], we_ref[...], preferred_element_type=jnp.float32)
    msg_ref[...] = jax.nn.sigmoid(f + ew[:, :D]) * jax.nn.softplus(s + ew[:, D:])


def _edge_msg(gd, gs, ea, we):
    bm = 2000
    return pl.pallas_call(
        _msg_body,
        grid=(E // bm,),
        in_specs=[
            pl.BlockSpec((bm, D), lambda i: (i, 0)),
            pl.BlockSpec((bm, D), lambda i: (i, 0)),
            pl.BlockSpec((bm, DE), lambda i: (i, 0)),
            pl.BlockSpec((DE, 2 * D), lambda i: (0, 0)),
        ],
        out_specs=pl.BlockSpec((bm, D), lambda i: (i, 0)),
        out_shape=jax.ShapeDtypeStruct((E, D), jnp.float32),
    )(gd, gs, ea, we)


@functools.partial(
    pl.kernel,
    out_type=jax.ShapeDtypeStruct((NC, N, D), jnp.float32),
    mesh=_mesh,
    scratch_types=[
        pltpu.VMEM((NCHS, CHS), jnp.int32),
        pltpu.VMEM((CHS, D), jnp.float32),
        pltpu.VMEM((ZR, D), jnp.float32),
        pltpu.VMEM_SHARED((N, D), jnp.float32),
        pltpu.SemaphoreType.DMA,
    ],
)
def _scatter_k(msg_hbm, dst_hbm, aggp_hbm, idxd, mbuf, zbuf, aggsh, sem):
    cid = lax.axis_index("c")
    sid = lax.axis_index("s")
    wid = sid * NC + cid
    pltpu.sync_copy(dst_hbm.at[wid], idxd)

    def zrow(i, _):
        for j in range(D // 16):
            zbuf[i, pl.ds(j * 16, 16)] = jnp.zeros((16,), jnp.float32)
        return 0

    lax.fori_loop(0, ZR, zrow, 0)

    @pl.when(sid < WTILES)
    def _init():
        for t in range(RPS // ZR):
            pltpu.sync_copy(zbuf, aggsh.at[pl.ds(sid * RPS + t * ZR, ZR)])

    plsc.subcore_barrier()

    def chunk(c, _):
        pltpu.sync_copy(msg_hbm.at[pl.ds(wid * EPW + c * CHS, CHS)], mbuf)
        pltpu.sync_copy(mbuf, aggsh.at[idxd.at[c]], add=True)
        return 0

    lax.fori_loop(0, NCHS, chunk, 0)
    plsc.subcore_barrier()

    @pl.when(sid < WTILES)
    def _writeout():
        pltpu.sync_copy(
            aggsh.at[pl.ds(sid * RPS, RPS)],
            aggp_hbm.at[cid, pl.ds(sid * RPS, RPS)],
        )


def _final_body(x_ref, ap_ref, b_ref, w1_ref, b1_ref, w2_ref, b2_ref, out_ref):
    h = jax.nn.sigmoid(x_ref[...] + ap_ref[0] + ap_ref[1])
    h = jax.nn.sigmoid(
        jnp.dot(h, w1_ref[...], preferred_element_type=jnp.float32) + b1_ref[...]
    )
    oh = (
        b_ref[...] == lax.broadcasted_iota(jnp.int32, (N, G), 1)
    ).astype(jnp.float32)
    pooled = lax.dot_general(
        oh, h, (((0,), (0,)), ((), ())), preferred_element_type=jnp.float32
    )
    out = jnp.dot(pooled, w2_ref[...], preferred_element_type=jnp.float32)
    out_ref[...] = jnp.maximum(out + b2_ref[...], 0.0)


def _final(x, aggp, batch2d, w1, b1, w2, b2):
    return pl.pallas_call(
        _final_body,
        out_shape=jax.ShapeDtypeStruct((G, 1), jnp.float32),
    )(x, aggp, batch2d, w1, b1, w2, b2)


def kernel(x, edge_index, edge_attr, batch, Wf, bf, Ws, bs, W1, b1, W2, b2):
    src = edge_index[0]
    dst = edge_index[1]
    wd = jnp.concatenate([Wf[:D], Ws[:D]], axis=1)
    wsr = jnp.concatenate([Wf[D : 2 * D], Ws[D : 2 * D]], axis=1)
    we = jnp.concatenate([Wf[2 * D :], Ws[2 * D :]], axis=1)
    bcat = jnp.concatenate([bf, bs]).reshape(1, 2 * D)
    td, ts = _tables(x, wd, wsr, bcat)
    tdp = lax.bitcast_convert_type(
        jnp.stack([td[:, :D], td[:, D:]], axis=-1), jnp.int32
    )
    tsp = lax.bitcast_convert_type(
        jnp.stack([ts[:, :D], ts[:, D:]], axis=-1), jnp.int32
    )
    dst3 = dst.reshape(NW, NCHUNK, CH)
    src3 = src.reshape(NW, NCHUNK, CH)
    gd, gs = _gather_k(tdp, tsp, dst3, src3)
    msg = _edge_msg(gd, gs, edge_attr, we)
    aggp = _scatter_k(msg, dst.reshape(NW, NCHS, CHS))
    return _final(
        x,
        aggp,
        batch.reshape(N, 1),
        W1,
        b1.reshape(1, H),
        W2,
        b2.reshape(1, 1),
    )
